# baseline (speedup 1.0000x reference)
import functools

import jax
import jax.numpy as jnp
from jax import lax
from jax.experimental import pallas as pl
from jax.experimental.pallas import tpu as pltpu
from jax.experimental.pallas import tpu_sc as plsc

B = 16384
N_CONT = 13
N_BINARY = 26
N_CAT = 26
VOCAB = 100000
D_F = 32
N_FEAT = N_CONT + N_BINARY + N_CAT

NC, NS = 2, 16
NW = NC * NS
SAMP_PER_W = B // NW
CHUNK_S = 32
N_CHUNKS = SAMP_PER_W // CHUNK_S
CIDX = CHUNK_S * N_CAT


def _body(cat_tbl_hbm, bin_tbl_hbm, idxc_hbm, idxb_hbm, xc_hbm, wb_hbm,
          out_hbm, idxc_v, idxb_v, rowsc_v, rowsb_v, cont_v, xc_v, wb_v,
          sem, out_sem):
    wid = lax.axis_index("s") * NC + lax.axis_index("c")
    base = wid * SAMP_PER_W

    pltpu.sync_copy(wb_hbm, wb_v)
    w_lo = wb_v[pl.ds(0, 16)]
    w_hi = wb_v[pl.ds(16, 16)]
    b_lo = wb_v[pl.ds(32, 16)]
    b_hi = wb_v[pl.ds(48, 16)]

    @pl.loop(0, N_CHUNKS)
    def _(k):
        s0 = base + k * CHUNK_S
        xo = s0 * N_CONT

        pltpu.sync_copy(idxc_hbm.at[pl.ds(s0 * N_CAT, CIDX)], idxc_v)
        pltpu.sync_copy(idxb_hbm.at[pl.ds(s0 * N_BINARY, CIDX)], idxb_v)
        pltpu.sync_copy(xc_hbm.at[pl.ds(xo, CHUNK_S * N_CONT)], xc_v)

        pltpu.async_copy(cat_tbl_hbm.at[idxc_v], rowsc_v, sem).wait()

        @pl.loop(0, CHUNK_S)
        def _(s):
            pltpu.async_copy(
                rowsc_v.at[pl.ds(s * N_CAT, N_CAT), :],
                out_hbm.at[s0 + s, pl.ds(N_CONT + N_BINARY, N_CAT), :],
                out_sem,
            )

        pltpu.async_copy(bin_tbl_hbm.at[idxb_v], rowsb_v, sem).wait()

        @pl.loop(0, CHUNK_S)
        def _(s):
            pltpu.async_copy(
                rowsb_v.at[pl.ds(s * N_BINARY, N_BINARY), :],
                out_hbm.at[s0 + s, pl.ds(N_CONT, N_BINARY), :],
                out_sem,
            )

        @pl.loop(0, CHUNK_S)
        def _(s):
            for i in range(N_CONT):
                pos = jnp.full((16,), s * N_CONT + i, dtype=jnp.int32)
                xv = plsc.load_gather(xc_v, [pos])
                cont_v[s, i, pl.ds(0, 16)] = xv * w_lo + b_lo
                cont_v[s, i, pl.ds(16, 16)] = xv * w_hi + b_hi

        pltpu.sync_copy(
            cont_v, out_hbm.at[pl.ds(s0, CHUNK_S), pl.ds(0, N_CONT), :]
        )

        pltpu.make_async_copy(
            cat_tbl_hbm.at[pl.ds(0, CIDX)], rowsc_v, out_sem
        ).wait()
        pltpu.make_async_copy(
            cat_tbl_hbm.at[pl.ds(0, CIDX)], rowsb_v, out_sem
        ).wait()


def kernel(x_cont, x_binary, x_cat, W_cont, b_cont, binary_tables, cat_tables):
    idxc = (
        x_cat.astype(jnp.int32)
        + (jnp.arange(N_CAT, dtype=jnp.int32) * VOCAB)[None, :]
    ).reshape(B * N_CAT)
    idxb = (
        x_binary.astype(jnp.int32)
        + (jnp.arange(N_BINARY, dtype=jnp.int32) * 2)[None, :]
    ).reshape(B * N_BINARY)
    xc_flat = x_cont.reshape(B * N_CONT)
    wb = jnp.concatenate([W_cont[0], b_cont])

    mesh = plsc.VectorSubcoreMesh(core_axis_name="c", subcore_axis_name="s")
    k = pl.kernel(
        _body,
        out_type=jax.ShapeDtypeStruct((B, N_FEAT, D_F), jnp.float32),
        mesh=mesh,
        scratch_types=[
            pltpu.VMEM((CIDX,), jnp.int32),
            pltpu.VMEM((CIDX,), jnp.int32),
            pltpu.VMEM((CIDX, D_F), jnp.float32),
            pltpu.VMEM((CIDX, D_F), jnp.float32),
            pltpu.VMEM((CHUNK_S, N_CONT, D_F), jnp.float32),
            pltpu.VMEM((CHUNK_S * N_CONT,), jnp.float32),
            pltpu.VMEM((2 * D_F,), jnp.float32),
            pltpu.SemaphoreType.DMA,
            pltpu.SemaphoreType.DMA,
        ],
        compiler_params=pltpu.CompilerParams(
            use_tc_tiling_on_sc=False, needs_layout_passes=False
        ),
    )
    return k(
        cat_tables.reshape(N_CAT * VOCAB, D_F),
        binary_tables.reshape(2 * N_BINARY, D_F),
        idxc,
        idxb,
        xc_flat,
        wb,
    )


# speedup vs baseline: 1.5295x; 1.5295x over previous
import functools

import jax
import jax.numpy as jnp
from jax import lax
from jax.experimental import pallas as pl
from jax.experimental.pallas import tpu as pltpu
from jax.experimental.pallas import tpu_sc as plsc

B = 16384
N_CONT = 13
N_BINARY = 26
N_CAT = 26
VOCAB = 100000
D_F = 32
N_FEAT = N_CONT + N_BINARY + N_CAT

NC, NS = 2, 16
NW = NC * NS
TOTAL_IDX = B * N_CAT
IDX_PER_W = TOTAL_IDX // NW
CHUNK = 1664

BB = 1024


def _gather_body(table_hbm, idx_hbm, out_hbm, idx_v, rows_v, sem):
    wid = lax.axis_index("s") * NC + lax.axis_index("c")
    base = wid * IDX_PER_W

    @pl.loop(0, IDX_PER_W, step=CHUNK)
    def _(off):
        pltpu.sync_copy(idx_hbm.at[pl.ds(base + off, CHUNK)], idx_v)
        pltpu.async_copy(table_hbm.at[idx_v], rows_v, sem).wait()
        pltpu.sync_copy(rows_v, out_hbm.at[pl.ds(base + off, CHUNK)])


def _cat_gather(table_flat, idx_flat):
    mesh = plsc.VectorSubcoreMesh(core_axis_name="c", subcore_axis_name="s")
    k = pl.kernel(
        _gather_body,
        out_type=jax.ShapeDtypeStruct((TOTAL_IDX, D_F), jnp.float32),
        mesh=mesh,
        scratch_types=[
            pltpu.VMEM((CHUNK,), jnp.int32),
            pltpu.VMEM((CHUNK, D_F), jnp.float32),
            pltpu.SemaphoreType.DMA,
        ],
        compiler_params=pltpu.CompilerParams(use_tc_tiling_on_sc=False),
    )
    return k(table_flat, idx_flat)


def _asm_body(xc_ref, xb_ref, wb_ref, t0_ref, dt_ref, g_ref, o_ref):
    wcol = wb_ref[:, 0:1]
    bcol = wb_ref[:, 1:2]
    for i in range(N_CONT):
        o_ref[i] = wcol * xc_ref[i:i + 1, :] + bcol
    for i in range(N_BINARY):
        o_ref[N_CONT + i] = (
            t0_ref[:, i:i + 1] + dt_ref[:, i:i + 1] * xb_ref[i:i + 1, :]
        )
    for i in range(N_CAT):
        o_ref[N_CONT + N_BINARY + i] = jnp.transpose(g_ref[i])


def _assemble(xc_t, xb_t, wb, t0_t, dt_t, g3):
    return pl.pallas_call(
        _asm_body,
        grid=(B // BB,),
        in_specs=[
            pl.BlockSpec((N_CONT, BB), lambda j: (0, j)),
            pl.BlockSpec((N_BINARY, BB), lambda j: (0, j)),
            pl.BlockSpec((D_F, 2), lambda j: (0, 0)),
            pl.BlockSpec((D_F, N_BINARY), lambda j: (0, 0)),
            pl.BlockSpec((D_F, N_BINARY), lambda j: (0, 0)),
            pl.BlockSpec((N_CAT, BB, D_F), lambda j: (0, j, 0)),
        ],
        out_specs=pl.BlockSpec((N_FEAT, D_F, BB), lambda j: (0, 0, j)),
        out_shape=jax.ShapeDtypeStruct((N_FEAT, D_F, B), jnp.float32),
    )(xc_t, xb_t, wb, t0_t, dt_t, g3)


def kernel(x_cont, x_binary, x_cat, W_cont, b_cont, binary_tables, cat_tables):
    xct = x_cat.T.astype(jnp.int32)
    idx = (
        xct + (jnp.arange(N_CAT, dtype=jnp.int32) * VOCAB)[:, None]
    ).reshape(TOTAL_IDX)
    table_flat = cat_tables.reshape(N_CAT * VOCAB, D_F)

    xc_t = x_cont.T
    xb_t = x_binary.T.astype(jnp.float32)
    wb = jnp.stack([W_cont[0], b_cont], axis=1)
    t0_t = binary_tables[:, 0, :].T
    dt_t = (binary_tables[:, 1, :] - binary_tables[:, 0, :]).T

    g = _cat_gather(table_flat, idx)
    g3 = g.reshape(N_CAT, B, D_F)

    out_t = _assemble(xc_t, xb_t, wb, t0_t, dt_t, g3)
    return jnp.transpose(out_t, (2, 0, 1))
